# baseline (speedup 1.0000x reference)
import jax
import jax.numpy as jnp
from jax.experimental import pallas as pl
from jax.experimental.pallas import tpu as pltpu

_BLOCK_ROWS = 2048


def _copy_body(x_ref, o_ref):
    o_ref[...] = x_ref[...]


def kernel(x):
    rows, cols = x.shape
    x2 = x.reshape(rows // 2, cols * 2)
    r2, c2 = x2.shape
    out = pl.pallas_call(
        _copy_body,
        grid=(r2 // _BLOCK_ROWS,),
        in_specs=[pl.BlockSpec((_BLOCK_ROWS, c2), lambda i: (i, 0))],
        out_specs=pl.BlockSpec((_BLOCK_ROWS, c2), lambda i: (i, 0)),
        out_shape=jax.ShapeDtypeStruct(x2.shape, x2.dtype),
    )(x2)
    return out.reshape(rows, cols)


# speedup vs baseline: 1.6797x; 1.6797x over previous
import jax
import jax.numpy as jnp
from jax.experimental import pallas as pl
from jax.experimental.pallas import tpu as pltpu

_N_CHUNKS = 8


def _copy_body(x_ref, o_ref, buf, in_sems, out_sems):
    rows = x_ref.shape[0]
    chunk = rows // _N_CHUNKS
    for i in range(_N_CHUNKS):
        sl = pl.ds(i * chunk, chunk)
        pltpu.make_async_copy(x_ref.at[sl, :], buf.at[sl, :], in_sems.at[i]).start()
    for i in range(_N_CHUNKS):
        sl = pl.ds(i * chunk, chunk)
        pltpu.make_async_copy(x_ref.at[sl, :], buf.at[sl, :], in_sems.at[i]).wait()
        pltpu.make_async_copy(buf.at[sl, :], o_ref.at[sl, :], out_sems.at[i]).start()
    for i in range(_N_CHUNKS):
        sl = pl.ds(i * chunk, chunk)
        pltpu.make_async_copy(buf.at[sl, :], o_ref.at[sl, :], out_sems.at[i]).wait()


def kernel(x):
    return pl.pallas_call(
        _copy_body,
        in_specs=[pl.BlockSpec(memory_space=pl.ANY)],
        out_specs=pl.BlockSpec(memory_space=pl.ANY),
        out_shape=jax.ShapeDtypeStruct(x.shape, x.dtype),
        scratch_shapes=[
            pltpu.VMEM(x.shape, x.dtype),
            pltpu.SemaphoreType.DMA((_N_CHUNKS,)),
            pltpu.SemaphoreType.DMA((_N_CHUNKS,)),
        ],
    )(x)
